# baseline (speedup 1.0000x reference)
import jax
import jax.numpy as jnp
from jax import lax
from jax.experimental import pallas as pl
from jax.experimental.pallas import tpu as pltpu

T = 128
HIDDEN = 1024
E = 64
TOPK = 2
G = 8
NTILES = 96
NPAIRS = T * TOPK


def _route_body(x_ref, Wg_ref, bg_ref, gate_ref, tok_ref, w_ref, te_ref):
    x = x_ref[...]
    logits = jnp.dot(x, Wg_ref[...], preferred_element_type=jnp.float32)
    logits = logits + bg_ref[...]
    m = jnp.max(logits, axis=1, keepdims=True)
    p = jnp.exp(logits - m)
    gate = p / jnp.sum(p, axis=1, keepdims=True)
    gate_ref[...] = gate

    iota_e = lax.broadcasted_iota(jnp.int32, (T, E), 1)
    m1 = jnp.max(gate, axis=1, keepdims=True)
    a1 = jnp.min(jnp.where(gate == m1, iota_e, E), axis=1, keepdims=True)
    sel1 = iota_e == a1
    gate2 = jnp.where(sel1, -1.0, gate)
    m2 = jnp.max(gate2, axis=1, keepdims=True)
    a2 = jnp.min(jnp.where(gate2 == m2, iota_e, E), axis=1, keepdims=True)
    sel2 = iota_e == a2

    P = jnp.concatenate([sel1.astype(jnp.float32), sel2.astype(jnp.float32)], axis=0)

    ones = jnp.ones((NPAIRS, 1), dtype=jnp.float32)
    n_col = lax.dot_general(P, ones, (((0,), (0,)), ((), ())))
    tiles_col = jnp.floor((n_col + (G - 1)) / G)
    er = lax.broadcasted_iota(jnp.int32, (E, E), 0)
    ec = lax.broadcasted_iota(jnp.int32, (E, E), 1)
    ltri_e = (ec < er).astype(jnp.float32)
    ts_col = jnp.dot(ltri_e, tiles_col,
                     preferred_element_type=jnp.float32)
    total = jnp.sum(tiles_col)

    pr = lax.broadcasted_iota(jnp.int32, (NPAIRS, NPAIRS), 0)
    pc = lax.broadcasted_iota(jnp.int32, (NPAIRS, NPAIRS), 1)
    ltri_p = (pc < pr).astype(jnp.float32)
    C = jnp.dot(ltri_p, P, preferred_element_type=jnp.float32)
    rank = jnp.sum(C * P, axis=1, keepdims=True)
    slot = G * jnp.dot(P, ts_col, preferred_element_type=jnp.float32) + rank

    tok = lax.broadcasted_iota(jnp.int32, (T, 1), 0).astype(jnp.float32)
    tok2 = jnp.concatenate([tok, tok], axis=0)
    wts = jnp.concatenate([m1, m2], axis=0)

    S = (lax.broadcasted_iota(jnp.int32, (NPAIRS, NTILES * G), 1).astype(jnp.float32)
         == slot).astype(jnp.float32)
    tok_ref[...] = lax.dot_general(S, tok2, (((0,), (0,)), ((), ())))
    w_ref[...] = lax.dot_general(S, wts, (((0,), (0,)), ((), ())))

    e_col = lax.broadcasted_iota(jnp.int32, (E, 1), 0).astype(jnp.float32)
    jt = lax.broadcasted_iota(jnp.int32, (E, NTILES), 1).astype(jnp.float32)
    ind = ((jt >= ts_col) & (jt < ts_col + tiles_col)
           & (n_col > 0)).astype(jnp.float32)
    te = lax.dot_general(ind, e_col, (((0,), (0,)), ((), ())))
    last_e = jnp.max(e_col * (n_col > 0).astype(jnp.float32))
    j_col = lax.broadcasted_iota(jnp.int32, (NTILES, 1), 0).astype(jnp.float32)
    te_ref[...] = te + jnp.where(j_col >= total, last_e, 0.0)


def _expert_body(te_sref, tok_ref, w_ref, x_ref, W1_ref, b1_ref, W2_ref, b2_ref,
                 out_ref):
    i = pl.program_id(0)

    @pl.when(i == 0)
    def _init():
        out_ref[...] = jnp.zeros_like(out_ref)

    wv = w_ref[0]

    @pl.when(jnp.sum(wv) > 0.0)
    def _tile():
        tok = tok_ref[0]
        iota_t = lax.broadcasted_iota(jnp.int32, (G, T), 1).astype(jnp.float32)
        oh = (tok == iota_t).astype(jnp.float32)
        xg = jnp.dot(oh, x_ref[...], preferred_element_type=jnp.float32)
        h = jnp.dot(xg, W1_ref[0], preferred_element_type=jnp.float32)
        h = jnp.maximum(h + b1_ref[0], 0.0)
        y = jnp.dot(h, W2_ref[0], preferred_element_type=jnp.float32)
        y = y + b2_ref[0]
        ohw = oh * wv
        out_ref[...] += lax.dot_general(ohw, y, (((0,), (0,)), ((), ())))


def kernel(x, Wg, bg, W1, b1, W2, b2):
    bg2 = bg.reshape(1, E)
    b1 = b1.reshape(E, 1, HIDDEN)
    b2 = b2.reshape(E, 1, HIDDEN)

    gate, tokf, wf, tef = pl.pallas_call(
        _route_body,
        in_specs=[
            pl.BlockSpec((T, HIDDEN), lambda: (0, 0)),
            pl.BlockSpec((HIDDEN, E), lambda: (0, 0)),
            pl.BlockSpec((1, E), lambda: (0, 0)),
        ],
        out_specs=[
            pl.BlockSpec((T, E), lambda: (0, 0)),
            pl.BlockSpec((NTILES * G, 1), lambda: (0, 0)),
            pl.BlockSpec((NTILES * G, 1), lambda: (0, 0)),
            pl.BlockSpec((NTILES, 1), lambda: (0, 0)),
        ],
        out_shape=[
            jax.ShapeDtypeStruct((T, E), jnp.float32),
            jax.ShapeDtypeStruct((NTILES * G, 1), jnp.float32),
            jax.ShapeDtypeStruct((NTILES * G, 1), jnp.float32),
            jax.ShapeDtypeStruct((NTILES, 1), jnp.float32),
        ],
    )(x, Wg, bg2)

    te = tef.reshape(NTILES).astype(jnp.int32)
    tok3 = tokf.reshape(NTILES, G, 1)
    w3 = wf.reshape(NTILES, G, 1)

    out = pl.pallas_call(
        _expert_body,
        grid_spec=pltpu.PrefetchScalarGridSpec(
            num_scalar_prefetch=1,
            grid=(NTILES,),
            in_specs=[
                pl.BlockSpec((1, G, 1), lambda i, te: (i, 0, 0)),
                pl.BlockSpec((1, G, 1), lambda i, te: (i, 0, 0)),
                pl.BlockSpec((T, HIDDEN), lambda i, te: (0, 0)),
                pl.BlockSpec((1, HIDDEN, HIDDEN), lambda i, te: (te[i], 0, 0)),
                pl.BlockSpec((1, 1, HIDDEN), lambda i, te: (te[i], 0, 0)),
                pl.BlockSpec((1, HIDDEN, HIDDEN), lambda i, te: (te[i], 0, 0)),
                pl.BlockSpec((1, 1, HIDDEN), lambda i, te: (te[i], 0, 0)),
            ],
            out_specs=pl.BlockSpec((T, HIDDEN), lambda i, te: (0, 0)),
        ),
        out_shape=jax.ShapeDtypeStruct((T, HIDDEN), jnp.float32),
    )(te, tok3, w3, x, W1, b1, W2, b2)
    return (out, gate)


# speedup vs baseline: 1.1013x; 1.1013x over previous
import jax
import jax.numpy as jnp
from jax import lax
from jax.experimental import pallas as pl
from jax.experimental.pallas import tpu as pltpu

T = 128
HIDDEN = 1024
E = 64
TOPK = 2


def _moe_body(x_ref, Wg_ref, bg_ref, W1_ref, b1_ref, W2_ref, b2_ref,
              out_ref, gate_ref, wmat_ref):
    e = pl.program_id(0)

    @pl.when(e == 0)
    def _gate():
        x = x_ref[...]
        logits = jnp.dot(x, Wg_ref[...], preferred_element_type=jnp.float32)
        logits = logits + bg_ref[...]
        m = jnp.max(logits, axis=1, keepdims=True)
        p = jnp.exp(logits - m)
        gate = p / jnp.sum(p, axis=1, keepdims=True)
        gate_ref[...] = gate

        iota_e = lax.broadcasted_iota(jnp.int32, (T, E), 1)
        m1 = jnp.max(gate, axis=1, keepdims=True)
        a1 = jnp.min(jnp.where(gate == m1, iota_e, E), axis=1, keepdims=True)
        sel1 = iota_e == a1
        gate2 = jnp.where(sel1, -1.0, gate)
        m2 = jnp.max(gate2, axis=1, keepdims=True)
        a2 = jnp.min(jnp.where(gate2 == m2, iota_e, E), axis=1, keepdims=True)
        sel2 = iota_e == a2
        wmat_ref[...] = jnp.where(sel1, m1, 0.0) + jnp.where(sel2, m2, 0.0)
        out_ref[...] = jnp.zeros_like(out_ref)

    onehot = (lax.broadcasted_iota(jnp.int32, (E, 1), 0) == e).astype(jnp.float32)
    col = jnp.dot(wmat_ref[...], onehot, preferred_element_type=jnp.float32)

    @pl.when(jnp.sum(col) > 0.0)
    def _expert():
        xb = x_ref[...].astype(jnp.bfloat16)
        h = jnp.dot(xb, W1_ref[0].astype(jnp.bfloat16),
                    preferred_element_type=jnp.float32)
        h = jnp.maximum(h + b1_ref[0], 0.0)
        y = jnp.dot(h.astype(jnp.bfloat16), W2_ref[0].astype(jnp.bfloat16),
                    preferred_element_type=jnp.float32)
        y = y + b2_ref[0]
        out_ref[...] += col * y


def kernel(x, Wg, bg, W1, b1, W2, b2):
    bg2 = bg.reshape(1, E)
    b1 = b1.reshape(E, 1, HIDDEN)
    b2 = b2.reshape(E, 1, HIDDEN)
    out, gate = pl.pallas_call(
        _moe_body,
        grid=(E,),
        in_specs=[
            pl.BlockSpec((T, HIDDEN), lambda e: (0, 0)),
            pl.BlockSpec((HIDDEN, E), lambda e: (0, 0)),
            pl.BlockSpec((1, E), lambda e: (0, 0)),
            pl.BlockSpec((1, HIDDEN, HIDDEN), lambda e: (e, 0, 0)),
            pl.BlockSpec((1, 1, HIDDEN), lambda e: (e, 0, 0)),
            pl.BlockSpec((1, HIDDEN, HIDDEN), lambda e: (e, 0, 0)),
            pl.BlockSpec((1, 1, HIDDEN), lambda e: (e, 0, 0)),
        ],
        out_specs=[
            pl.BlockSpec((T, HIDDEN), lambda e: (0, 0)),
            pl.BlockSpec((T, E), lambda e: (0, 0)),
        ],
        out_shape=[
            jax.ShapeDtypeStruct((T, HIDDEN), jnp.float32),
            jax.ShapeDtypeStruct((T, E), jnp.float32),
        ],
        scratch_shapes=[pltpu.VMEM((T, E), jnp.float32)],
    )(x, Wg, bg2, W1, b1, W2, b2)
    return (out, gate)
